# baseline (speedup 1.0000x reference)
import functools

import jax
import jax.numpy as jnp
from jax import lax
from jax.experimental import pallas as pl
from jax.experimental.pallas import tpu as pltpu
from jax.experimental.pallas import tpu_sc as plsc

VOCAB = 1000000
D = 64
N_TOK = 20
B = 1024
S = 200
OUT_S = N_TOK + S

NC = 2
NS = 16
NW = NC * NS
BPW = B // NW
CH = 100
NCH = S // CH


def _body(tokens_hbm, wte_hbm, learned_hbm, out_hbm, idx_v, buf_v, sem):
    wid = lax.axis_index("s") * NC + lax.axis_index("c")
    base = wid * BPW
    pltpu.sync_copy(learned_hbm, buf_v.at[pl.ds(0, N_TOK)])

    def one_batch(i, carry):
        b = base + i
        pltpu.sync_copy(tokens_hbm.at[b], idx_v)
        for j in range(NCH):
            pltpu.async_copy(
                wte_hbm.at[idx_v.at[j]],
                buf_v.at[pl.ds(N_TOK + j * CH, CH)],
                sem,
            ).wait()
        pltpu.sync_copy(buf_v, out_hbm.at[b])
        return carry

    lax.fori_loop(0, BPW, one_batch, 0)


@functools.partial(jax.jit)
def kernel(tokens, wte_weight, learned_embedding):
    tokens3 = tokens.reshape(B, NCH, CH).astype(jnp.int32)
    mesh = plsc.VectorSubcoreMesh(core_axis_name="c", subcore_axis_name="s")
    k = pl.kernel(
        _body,
        mesh=mesh,
        out_type=jax.ShapeDtypeStruct((B, OUT_S, D), jnp.float32),
        scratch_types=[
            pltpu.VMEM((NCH, CH), jnp.int32),
            pltpu.VMEM((OUT_S, D), jnp.float32),
            pltpu.SemaphoreType.DMA,
        ],
        compiler_params=pltpu.CompilerParams(use_tc_tiling_on_sc=False),
    )
    return k(tokens3, wte_weight, learned_embedding)


# speedup vs baseline: 1.0614x; 1.0614x over previous
import functools

import jax
import jax.numpy as jnp
from jax import lax
from jax.experimental import pallas as pl
from jax.experimental.pallas import tpu as pltpu
from jax.experimental.pallas import tpu_sc as plsc

VOCAB = 1000000
D = 64
N_TOK = 20
B = 1024
S = 200
OUT_S = N_TOK + S

NC = 2
NS = 16
NW = NC * NS
BPW = B // NW
CH = 100
NCH = S // CH
G = 4
NG = BPW // G


def _body(tokens_hbm, wte_hbm, learned_hbm, out_hbm,
          idx_v, buf_a, buf_b, gsem_a, gsem_b, wsem_a, wsem_b):
    wid = lax.axis_index("s") * NC + lax.axis_index("c")
    base = wid * BPW

    pltpu.sync_copy(tokens_hbm.at[pl.ds(base, BPW)], idx_v)

    for buf in (buf_a, buf_b):
        for k in range(G):
            pltpu.sync_copy(learned_hbm, buf.at[k, pl.ds(0, N_TOK)])

    bufs = ((buf_a, gsem_a, wsem_a), (buf_b, gsem_b, wsem_b))

    def issue_gathers(g, buf, gsem):
        descs = []
        for k in range(G):
            i = g * G + k
            for j in range(NCH):
                descs.append(pltpu.async_copy(
                    wte_hbm.at[idx_v.at[i, j]],
                    buf.at[k, pl.ds(N_TOK + j * CH, CH)],
                    gsem,
                ))
        return descs

    pending_g = {0: issue_gathers(0, buf_a, gsem_a), 1: None}
    pending_w = {0: None, 1: None}

    for g in range(NG):
        p = g % 2
        buf, gsem, wsem = bufs[p]
        for dsc in pending_g[p]:
            dsc.wait()
        pending_w[p] = pltpu.async_copy(
            buf, out_hbm.at[pl.ds(base + g * G, G)], wsem)
        if g + 1 < NG:
            q = 1 - p
            if pending_w[q] is not None:
                pending_w[q].wait()
                pending_w[q] = None
            pending_g[q] = issue_gathers(g + 1, bufs[q][0], bufs[q][1])

    for p in (0, 1):
        if pending_w[p] is not None:
            pending_w[p].wait()


@functools.partial(jax.jit)
def kernel(tokens, wte_weight, learned_embedding):
    tokens3 = tokens.reshape(B, NCH, CH).astype(jnp.int32)
    mesh = plsc.VectorSubcoreMesh(core_axis_name="c", subcore_axis_name="s")
    k = pl.kernel(
        _body,
        mesh=mesh,
        out_type=jax.ShapeDtypeStruct((B, OUT_S, D), jnp.float32),
        scratch_types=[
            pltpu.VMEM((BPW, NCH, CH), jnp.int32),
            pltpu.VMEM((G, OUT_S, D), jnp.float32),
            pltpu.VMEM((G, OUT_S, D), jnp.float32),
            pltpu.SemaphoreType.DMA,
            pltpu.SemaphoreType.DMA,
            pltpu.SemaphoreType.DMA,
            pltpu.SemaphoreType.DMA,
        ],
        compiler_params=pltpu.CompilerParams(use_tc_tiling_on_sc=False),
    )
    return k(tokens3, wte_weight, learned_embedding)
